# Initial kernel scaffold; baseline (speedup 1.0000x reference)
import functools

import jax
import jax.numpy as jnp
from jax import lax
from jax.experimental import pallas as pl
from jax.experimental.pallas import tpu as pltpu
from jax.experimental.pallas import tpu_sc as plsc

N = 10000
D = 128
DF = 256
E = 320000
NC = 2
NS = 16
CHUNK = 128
J = -(-E // (NC * NS * CHUNK))
EP = J * NC * NS * CHUNK
VPAD = 10240
RPT = VPAD // NS
RC = RPT // CHUNK

_mesh = plsc.VectorSubcoreMesh(core_axis_name="c", subcore_axis_name="s")


@functools.partial(
    pl.kernel,
    out_type=jax.ShapeDtypeStruct((NC, VPAD, 16), jnp.float32),
    mesh=_mesh,
    scratch_types=[
        pltpu.VMEM((J, CHUNK), jnp.int32),
        pltpu.VMEM((CHUNK, 16), jnp.float32),
        pltpu.VMEM_SHARED((VPAD, 16), jnp.float32),
    ],
)
def _deg_kernel(dst_hbm, out_hbm, dstv, buf, acc):
    c = lax.axis_index("c")
    s = lax.axis_index("s")
    pltpu.sync_copy(dst_hbm.at[c, s], dstv)

    def zero_row(r, carry):
        buf[r, :] = jnp.zeros((16,), jnp.float32)
        return carry
    lax.fori_loop(0, CHUNK, zero_row, 0)
    for t in range(RC):
        pltpu.sync_copy(buf, acc.at[pl.ds(s * RPT + t * CHUNK, CHUNK)])

    def ones_row(r, carry):
        buf[r, :] = jnp.ones((16,), jnp.float32)
        return carry
    lax.fori_loop(0, CHUNK, ones_row, 0)
    plsc.subcore_barrier()

    def body(j, carry):
        pltpu.sync_copy(buf, acc.at[dstv.at[j]], add=True)
        return carry
    lax.fori_loop(0, J, body, 0)
    plsc.subcore_barrier()

    for t in range(RC):
        rows = pl.ds(s * RPT + t * CHUNK, CHUNK)
        pltpu.sync_copy(acc.at[rows], out_hbm.at[c, rows])


@functools.partial(
    pl.kernel,
    out_type=jax.ShapeDtypeStruct((NC, VPAD, D), jnp.float32),
    mesh=_mesh,
    scratch_types=[
        pltpu.VMEM((J, CHUNK), jnp.int32),
        pltpu.VMEM((J, CHUNK), jnp.int32),
        pltpu.VMEM((CHUNK, D), jnp.float32),
        pltpu.VMEM((CHUNK, D), jnp.float32),
        pltpu.VMEM_SHARED((VPAD, D), jnp.float32),
        pltpu.SemaphoreType.DMA,
        pltpu.SemaphoreType.DMA,
    ],
)
def _agg_kernel(u_hbm, src_hbm, dst_hbm, out_hbm, srcv, dstv, buf0, buf1,
                acc, sem0, sem1):
    c = lax.axis_index("c")
    s = lax.axis_index("s")
    pltpu.sync_copy(src_hbm.at[c, s], srcv)
    pltpu.sync_copy(dst_hbm.at[c, s], dstv)

    def zero_row(r, carry):
        for i in range(D // 16):
            buf0[r, pl.ds(i * 16, 16)] = jnp.zeros((16,), jnp.float32)
        return carry
    lax.fori_loop(0, CHUNK, zero_row, 0)
    for t in range(RC):
        pltpu.sync_copy(buf0, acc.at[pl.ds(s * RPT + t * CHUNK, CHUNK)])
    plsc.subcore_barrier()

    pltpu.async_copy(u_hbm.at[srcv.at[0]], buf0, sem0)

    def body(j2, carry):
        for p, b, sm, nb, nsm in ((0, buf0, sem0, buf1, sem1),
                                  (1, buf1, sem1, buf0, sem0)):
            j = j2 * 2 + p
            pltpu.make_async_copy(u_hbm.at[srcv.at[j]], b, sm).wait()
            pltpu.async_copy(u_hbm.at[srcv.at[j + 1]], nb, nsm)
            pltpu.sync_copy(b, acc.at[dstv.at[j]], add=True)
        return carry
    lax.fori_loop(0, (J - 1) // 2, body, 0)
    j = J - 1
    blast, slast = (buf0, sem0) if j % 2 == 0 else (buf1, sem1)
    pltpu.make_async_copy(u_hbm.at[srcv.at[j]], blast, slast).wait()
    pltpu.sync_copy(blast, acc.at[dstv.at[j]], add=True)
    plsc.subcore_barrier()

    for t in range(RC):
        rows = pl.ds(s * RPT + t * CHUNK, CHUNK)
        pltpu.sync_copy(acc.at[rows], out_hbm.at[c, rows])


BLK = 1000
GRID = N // BLK


def _prep_body(d0_ref, d1_ref, x_ref, dinv_ref, u1_ref):
    deg = d0_ref[0, :, 0:1] + d1_ref[0, :, 0:1] + 1.0
    dinvb = jnp.broadcast_to(lax.rsqrt(deg), (BLK, D))
    dinv_ref[...] = dinvb
    u1_ref[...] = x_ref[...] * dinvb


def _layer_body(s0_ref, s1_ref, u_ref, dv_ref, w_ref, b_ref, unext_ref, *,
                relu):
    dv = dv_ref[...]
    agg = (s0_ref[0] + s1_ref[0] + u_ref[...]) * dv
    h = jnp.dot(agg, w_ref[...], preferred_element_type=jnp.float32) + b_ref[...]
    if relu:
        h = jnp.maximum(h, 0.0)
    unext_ref[...] = h * dv


def _final_body(s0_ref, s1_ref, u_ref, dv_ref, w5_ref, b5_ref, wl_ref,
                bl_ref, out_ref):
    agg = (s0_ref[0] + s1_ref[0] + u_ref[...]) * dv_ref[...]
    h5 = jnp.dot(agg, w5_ref[...], preferred_element_type=jnp.float32) + b5_ref[...]
    out_ref[...] = (jnp.dot(h5, wl_ref[...], preferred_element_type=jnp.float32)
                    + bl_ref[...])


def _row_spec(width):
    return pl.BlockSpec((BLK, width), lambda i: (i, 0))


def _part_spec(part, width):
    return pl.BlockSpec((1, BLK, width), lambda i, _p=part: (_p, i, 0))


def _full_spec(shape):
    nd = len(shape)
    return pl.BlockSpec(shape, lambda i: (0,) * nd)


_prep = pl.pallas_call(
    _prep_body,
    grid=(GRID,),
    in_specs=[_part_spec(0, 16), _part_spec(1, 16), _row_spec(D)],
    out_specs=[_row_spec(D), _row_spec(D)],
    out_shape=[jax.ShapeDtypeStruct((N, D), jnp.float32),
               jax.ShapeDtypeStruct((N, D), jnp.float32)],
)


def _make_layer(relu):
    return pl.pallas_call(
        functools.partial(_layer_body, relu=relu),
        grid=(GRID,),
        in_specs=[_part_spec(0, D), _part_spec(1, D), _row_spec(D),
                  _row_spec(D), _full_spec((D, D)), _full_spec((1, D))],
        out_specs=_row_spec(D),
        out_shape=jax.ShapeDtypeStruct((N, D), jnp.float32),
    )


_layer_relu = _make_layer(True)
_layer_lin = _make_layer(False)

_final = pl.pallas_call(
    _final_body,
    grid=(GRID,),
    in_specs=[_part_spec(0, D), _part_spec(1, D), _row_spec(D), _row_spec(D),
              _full_spec((D, DF)), _full_spec((1, DF)),
              _full_spec((DF, DF)), _full_spec((1, DF))],
    out_specs=_row_spec(DF),
    out_shape=jax.ShapeDtypeStruct((N, DF), jnp.float32),
)


def kernel(x, edge_index, W1, b1, W2, b2, W3, b3, W4, b4, W5, b5, Wlin, blin):
    ei = edge_index.astype(jnp.int32)
    pad = EP - E
    src = jnp.concatenate([ei[0], jnp.zeros((pad,), jnp.int32)])
    dst = jnp.concatenate([ei[1], jnp.full((pad,), VPAD - 1, jnp.int32)])
    src_r = src.reshape(NC, NS, J, CHUNK)
    dst_r = dst.reshape(NC, NS, J, CHUNK)

    degp = _deg_kernel(dst_r)
    dinvb, u = _prep(degp, degp, x)

    for w, b, layer in ((W1, b1, _layer_relu), (W2, b2, _layer_lin),
                        (W3, b3, _layer_lin), (W4, b4, _layer_lin)):
        sp = _agg_kernel(u, src_r, dst_r)
        u = layer(sp, sp, u, dinvb, w, b.reshape(1, D))

    sp = _agg_kernel(u, src_r, dst_r)
    w5p = jnp.pad(W5, ((0, 0), (0, DF - W5.shape[1])))
    b5p = jnp.pad(b5, (0, DF - b5.shape[0])).reshape(1, DF)
    wlp = jnp.pad(Wlin, ((0, DF - Wlin.shape[0]), (0, DF - Wlin.shape[1])))
    blp = jnp.pad(blin, (0, DF - blin.shape[0])).reshape(1, DF)
    out = _final(sp, sp, u, dinvb, w5p, b5p, wlp, blp)
    return out[:, :Wlin.shape[1]]


# speedup vs baseline: 9.3266x; 9.3266x over previous
import functools

import jax
import jax.numpy as jnp
from jax import lax
from jax.experimental import pallas as pl
from jax.experimental.pallas import tpu as pltpu
from jax.experimental.pallas import tpu_sc as plsc

N = 10000
D = 128
DH = 64
DF = 256
E = 320000
NC = 2
NS = 16
CHUNK = 128
J = -(-E // (NC * NS * CHUNK))
EP = J * NC * NS * CHUNK
VPAD = 10240
RPT = VPAD // NS
RC = RPT // CHUNK


@functools.cache
def _make_deg_kernel():
    mesh = plsc.VectorSubcoreMesh(core_axis_name="c", subcore_axis_name="s",
                                  num_cores=NC, num_subcores=NS)
    return functools.partial(
        pl.kernel,
        out_type=jax.ShapeDtypeStruct((NC, VPAD, 16), jnp.float32),
        mesh=mesh,
        scratch_types=[
            pltpu.VMEM((J, CHUNK), jnp.int32),
            pltpu.VMEM((CHUNK, 16), jnp.float32),
            pltpu.VMEM_SHARED((VPAD, 16), jnp.float32),
        ],
    )(_deg_body)


def _deg_body(dst_hbm, out_hbm, dstv, buf, acc):
    c = lax.axis_index("c")
    s = lax.axis_index("s")
    pltpu.sync_copy(dst_hbm.at[c, s], dstv)

    def zero_row(r, carry):
        buf[r, :] = jnp.zeros((16,), jnp.float32)
        return carry
    lax.fori_loop(0, CHUNK, zero_row, 0)
    for t in range(RC):
        pltpu.sync_copy(buf, acc.at[pl.ds(s * RPT + t * CHUNK, CHUNK)])

    def ones_row(r, carry):
        buf[r, :] = jnp.ones((16,), jnp.float32)
        return carry
    lax.fori_loop(0, CHUNK, ones_row, 0)
    plsc.subcore_barrier()

    def body(j, carry):
        pltpu.sync_copy(buf, acc.at[dstv.at[j]], add=True)
        return carry
    lax.fori_loop(0, J, body, 0)
    plsc.subcore_barrier()

    for t in range(RC):
        rows = pl.ds(s * RPT + t * CHUNK, CHUNK)
        pltpu.sync_copy(acc.at[rows], out_hbm.at[c, rows])


@functools.cache
def _make_agg_kernel():
    mesh = plsc.VectorSubcoreMesh(core_axis_name="c", subcore_axis_name="s",
                                  num_cores=NC, num_subcores=NS)
    return functools.partial(
        pl.kernel,
        out_type=jax.ShapeDtypeStruct((2, NC, VPAD, DH), jnp.float32),
        mesh=mesh,
        scratch_types=[
            pltpu.VMEM((J, CHUNK), jnp.int32),
            pltpu.VMEM((J, CHUNK), jnp.int32),
            pltpu.VMEM((CHUNK, DH), jnp.float32),
            pltpu.VMEM((CHUNK, DH), jnp.float32),
            pltpu.VMEM_SHARED((VPAD, DH), jnp.float32),
            pltpu.SemaphoreType.DMA,
            pltpu.SemaphoreType.DMA,
        ],
        compiler_params=pltpu.CompilerParams(use_tc_tiling_on_sc=False),
    )(_agg_body)


def _agg_body(ulo_hbm, uhi_hbm, src_hbm, dst_hbm, out_hbm, srcv, dstv,
              buf0, buf1, acc, sem0, sem1):
    c = lax.axis_index("c")
    s = lax.axis_index("s")
    pltpu.sync_copy(src_hbm.at[c, s], srcv)
    pltpu.sync_copy(dst_hbm.at[c, s], dstv)

    for half, u_hbm in ((0, ulo_hbm), (1, uhi_hbm)):
        def zero_row(r, carry):
            for i in range(DH // 16):
                buf0[r, pl.ds(i * 16, 16)] = jnp.zeros((16,), jnp.float32)
            return carry
        lax.fori_loop(0, CHUNK, zero_row, 0)
        for t in range(RC):
            pltpu.sync_copy(buf0, acc.at[pl.ds(s * RPT + t * CHUNK, CHUNK)])
        plsc.subcore_barrier()

        pltpu.async_copy(u_hbm.at[srcv.at[0]], buf0, sem0)

        def body(j2, carry):
            for p, b, sm, nb, nsm in ((0, buf0, sem0, buf1, sem1),
                                      (1, buf1, sem1, buf0, sem0)):
                j = j2 * 2 + p
                pltpu.make_async_copy(u_hbm.at[srcv.at[j]], b, sm).wait()
                pltpu.async_copy(u_hbm.at[srcv.at[j + 1]], nb, nsm)
                pltpu.sync_copy(b, acc.at[dstv.at[j]], add=True)
            return carry
        lax.fori_loop(0, (J - 1) // 2, body, 0)
        j = J - 1
        blast, slast = (buf0, sem0) if j % 2 == 0 else (buf1, sem1)
        pltpu.make_async_copy(u_hbm.at[srcv.at[j]], blast, slast).wait()
        pltpu.sync_copy(blast, acc.at[dstv.at[j]], add=True)
        plsc.subcore_barrier()

        for t in range(RC):
            rows = pl.ds(s * RPT + t * CHUNK, CHUNK)
            pltpu.sync_copy(acc.at[rows], out_hbm.at[half, c, rows])
        plsc.subcore_barrier()


BLK = 1000
GRID = N // BLK


def _prep_body(d0_ref, d1_ref, x_ref, dinv_ref, ulo_ref, uhi_ref):
    deg = d0_ref[0, :, 0:1] + d1_ref[0, :, 0:1] + 1.0
    dinvb = jnp.broadcast_to(lax.rsqrt(deg), (BLK, DH))
    dinv_ref[...] = dinvb
    ulo_ref[...] = x_ref[:, :DH] * dinvb
    uhi_ref[...] = x_ref[:, DH:] * dinvb


def _halves(slo0, slo1, shi0, shi1, ulo, uhi, dv):
    agg_lo = (slo0[0, 0] + slo1[0, 0] + ulo[...]) * dv
    agg_hi = (shi0[0, 0] + shi1[0, 0] + uhi[...]) * dv
    return agg_lo, agg_hi


def _layer_body(slo0, slo1, shi0, shi1, u_lo, u_hi, dv_ref, w_ref, b_ref,
                unlo_ref, unhi_ref, *, relu):
    dv = dv_ref[...]
    agg_lo, agg_hi = _halves(slo0, slo1, shi0, shi1, u_lo, u_hi, dv)
    h = (jnp.dot(agg_lo, w_ref[:DH, :], preferred_element_type=jnp.float32)
         + jnp.dot(agg_hi, w_ref[DH:, :], preferred_element_type=jnp.float32)
         + b_ref[...])
    if relu:
        h = jnp.maximum(h, 0.0)
    unlo_ref[...] = h[:, :DH] * dv
    unhi_ref[...] = h[:, DH:] * dv


def _final_body(slo0, slo1, shi0, shi1, u_lo, u_hi, dv_ref, w5_ref, b5_ref,
                wl_ref, bl_ref, out_ref):
    agg_lo, agg_hi = _halves(slo0, slo1, shi0, shi1, u_lo, u_hi, dv_ref[...])
    h5 = (jnp.dot(agg_lo, w5_ref[:DH, :], preferred_element_type=jnp.float32)
          + jnp.dot(agg_hi, w5_ref[DH:, :], preferred_element_type=jnp.float32)
          + b5_ref[...])
    out_ref[...] = (jnp.dot(h5, wl_ref[...], preferred_element_type=jnp.float32)
                    + bl_ref[...])


def _row_spec(width):
    return pl.BlockSpec((BLK, width), lambda i: (i, 0))


def _deg_spec(part):
    return pl.BlockSpec((1, BLK, 16), lambda i, _p=part: (_p, i, 0))


def _part_spec(half, part):
    return pl.BlockSpec((1, 1, BLK, DH),
                        lambda i, _h=half, _p=part: (_h, _p, i, 0))


def _full_spec(shape):
    nd = len(shape)
    return pl.BlockSpec(shape, lambda i: (0,) * nd)


_prep = pl.pallas_call(
    _prep_body,
    grid=(GRID,),
    in_specs=[_deg_spec(0), _deg_spec(1), _row_spec(D)],
    out_specs=[_row_spec(DH), _row_spec(DH), _row_spec(DH)],
    out_shape=[jax.ShapeDtypeStruct((N, DH), jnp.float32)] * 3,
)

_SP_SPECS = [_part_spec(0, 0), _part_spec(0, 1), _part_spec(1, 0),
             _part_spec(1, 1)]


def _make_layer(relu):
    return pl.pallas_call(
        functools.partial(_layer_body, relu=relu),
        grid=(GRID,),
        in_specs=_SP_SPECS + [_row_spec(DH), _row_spec(DH), _row_spec(DH),
                              _full_spec((D, D)), _full_spec((1, D))],
        out_specs=[_row_spec(DH), _row_spec(DH)],
        out_shape=[jax.ShapeDtypeStruct((N, DH), jnp.float32)] * 2,
    )


_layer_relu = _make_layer(True)
_layer_lin = _make_layer(False)

_final = pl.pallas_call(
    _final_body,
    grid=(GRID,),
    in_specs=_SP_SPECS + [_row_spec(DH), _row_spec(DH), _row_spec(DH),
                          _full_spec((D, DF)), _full_spec((1, DF)),
                          _full_spec((DF, DF)), _full_spec((1, DF))],
    out_specs=_row_spec(DF),
    out_shape=jax.ShapeDtypeStruct((N, DF), jnp.float32),
)


def kernel(x, edge_index, W1, b1, W2, b2, W3, b3, W4, b4, W5, b5, Wlin, blin):
    ei = edge_index.astype(jnp.int32)
    pad = EP - E
    src = jnp.concatenate([ei[0], jnp.zeros((pad,), jnp.int32)])
    dst = jnp.concatenate([ei[1], jnp.full((pad,), VPAD - 1, jnp.int32)])
    src_r = src.reshape(NC, NS, J, CHUNK)
    dst_r = dst.reshape(NC, NS, J, CHUNK)

    degp = _make_deg_kernel()(dst_r)
    dv, ulo, uhi = _prep(degp, degp, x)

    agg = _make_agg_kernel()
    for w, b, layer in ((W1, b1, _layer_relu), (W2, b2, _layer_lin),
                        (W3, b3, _layer_lin), (W4, b4, _layer_lin)):
        sp = agg(ulo, uhi, src_r, dst_r)
        ulo, uhi = layer(sp, sp, sp, sp, ulo, uhi, dv, w, b.reshape(1, D))

    sp = agg(ulo, uhi, src_r, dst_r)
    w5p = jnp.pad(W5, ((0, 0), (0, DF - W5.shape[1])))
    b5p = jnp.pad(b5, (0, DF - b5.shape[0])).reshape(1, DF)
    wlp = jnp.pad(Wlin, ((0, DF - Wlin.shape[0]), (0, DF - Wlin.shape[1])))
    blp = jnp.pad(blin, (0, DF - blin.shape[0])).reshape(1, DF)
    out = _final(sp, sp, sp, sp, ulo, uhi, dv, w5p, b5p, wlp, blp)
    return out[:, :Wlin.shape[1]]
